# baseline (speedup 1.0000x reference)
import functools

import jax
import jax.numpy as jnp
from jax import lax
from jax.experimental import pallas as pl
from jax.experimental.pallas import tpu as pltpu
from jax.experimental.pallas import tpu_sc as plsc

_SIZE = 512
_L = 16
_NC = 2
_NS = 16


@functools.lru_cache(maxsize=None)
def _make_project(T, P, NV, NU):
    tpad = -(-T // _L) * _L
    nchunk = P // _L
    OVX, OVY, OVZ, OUU, OUV = 0, NV, 2 * NV, 3 * NV, 3 * NV + NU
    nf = 3 * NV + 2 * NU
    ni = 6 * tpad

    mesh = plsc.VectorSubcoreMesh(
        core_axis_name="c", subcore_axis_name="s", num_cores=_NC, num_subcores=_NS
    )

    @functools.partial(
        pl.kernel,
        out_type=jax.ShapeDtypeStruct((nchunk, 3, _L), jnp.float32),
        mesh=mesh,
        compiler_params=pltpu.CompilerParams(needs_layout_passes=False),
        scratch_types=[
            pltpu.VMEM((nf,), jnp.float32),
            pltpu.VMEM((ni,), jnp.int32),
            pltpu.VMEM((2, _L), jnp.float32),
            pltpu.VMEM((tpad * 32,), jnp.float32),
            pltpu.VMEM((3, _L), jnp.float32),
            pltpu.SemaphoreType.DMA,
            pltpu.SemaphoreType.DMA,
            pltpu.SemaphoreType.DMA,
        ],
    )
    def project(cf_hbm, ci_hbm, pts_hbm, out_hbm, cf, ci, ptsv, tab, obuf,
                s0, s1, s2):
        wid = lax.axis_index("s") * _NC + lax.axis_index("c")

        @pl.when(wid < nchunk)
        def _():
            d0 = pltpu.async_copy(cf_hbm, cf, s0)
            d1 = pltpu.async_copy(ci_hbm, ci, s1)
            d2 = pltpu.async_copy(pts_hbm.at[wid], ptsv, s2)
            d0.wait()
            d1.wait()
            d2.wait()

            for g in range(tpad // _L):
                o = g * _L
                fi0 = ci[pl.ds(o, _L)]
                fi1 = ci[pl.ds(tpad + o, _L)]
                fi2 = ci[pl.ds(2 * tpad + o, _L)]
                uf0 = ci[pl.ds(3 * tpad + o, _L)]
                uf1 = ci[pl.ds(4 * tpad + o, _L)]
                uf2 = ci[pl.ds(5 * tpad + o, _L)]

                ax = plsc.load_gather(cf, [fi0 + OVX])
                ay = plsc.load_gather(cf, [fi0 + OVY])
                az = plsc.load_gather(cf, [fi0 + OVZ])
                bx = plsc.load_gather(cf, [fi1 + OVX])
                by = plsc.load_gather(cf, [fi1 + OVY])
                bz = plsc.load_gather(cf, [fi1 + OVZ])
                cx = plsc.load_gather(cf, [fi2 + OVX])
                cy = plsc.load_gather(cf, [fi2 + OVY])
                cz = plsc.load_gather(cf, [fi2 + OVZ])
                ua = plsc.load_gather(cf, [uf0 + OUU])
                va = plsc.load_gather(cf, [uf0 + OUV])
                ub = plsc.load_gather(cf, [uf1 + OUU])
                vb = plsc.load_gather(cf, [uf1 + OUV])
                uc = plsc.load_gather(cf, [uf2 + OUU])
                vc = plsc.load_gather(cf, [uf2 + OUV])

                cross = (bx - ax) * (cy - ay) - (by - ay) * (cx - ax)
                w = 0.5 * cross
                valid = (cross > 0.0) & (w >= 1e-9)
                wsafe = jnp.where(w == 0.0, 1.0, w)
                h = 0.5 / wsafe

                def edge(qx, qy, rx, ry):
                    return ((qx * ry - qy * rx) * h,
                            (qy - ry) * h,
                            (rx - qx) * h)

                w1c0, w1cx, w1cy = edge(bx, by, cx, cy)
                w2c0, w2cx, w2cy = edge(cx, cy, ax, ay)
                a0c0, a0cx, a0cy = edge(ax, ay, bx, by)

                inf = jnp.float32(jnp.inf)
                xmin = jnp.where(valid, jnp.minimum(jnp.minimum(ax, bx), cx), inf)
                xmax = jnp.where(valid, jnp.maximum(jnp.maximum(ax, bx), cx), -inf)
                ymin = jnp.minimum(jnp.minimum(ay, by), cy)
                ymax = jnp.maximum(jnp.maximum(ay, by), cy)

                zia = 1.0 / az
                zib = 1.0 / bz
                zic = 1.0 / cz
                rows = [
                    xmin, xmax, ymin, ymax,
                    w1c0, w1cx, w1cy,
                    w2c0, w2cx, w2cy,
                    a0c0, a0cx, a0cy,
                    ua * zia, ub * zib, uc * zic,
                    va * zia, vb * zib, vc * zic,
                    zia, zib, zic,
                ]
                lanes = lax.broadcasted_iota(jnp.int32, (_L,), 0) + o
                for k, val in enumerate(rows):
                    plsc.store_scatter(tab, [lanes * 32 + k], val)

            px = ptsv[0, :] / (_SIZE - 1) * 2.0 - 1.0
            py = (_SIZE - ptsv[1, :]) / (_SIZE - 1) * 2.0 - 1.0

            bs = jnp.full((_L,), -jnp.inf, jnp.float32)
            bu = jnp.zeros((_L,), jnp.float32)
            bv = jnp.zeros((_L,), jnp.float32)
            for t in range(T):
                ca = tab[pl.ds(t * 32, _L)]
                cb = tab[pl.ds(t * 32 + _L, _L)]
                inb = ((px >= ca[0]) & (px <= ca[1])
                       & (py >= ca[2]) & (py <= ca[3]))
                w1 = ca[4] + ca[5] * px + ca[6] * py
                w2 = ca[7] + ca[8] * px + ca[9] * py
                a0 = ca[10] + ca[11] * px + ca[12] * py
                w3 = 1.0 - w1 - w2
                zi = w1 * cb[3] + w2 * cb[4] + w3 * cb[5]
                ptz = 1.0 / zi
                uu = (w1 * ca[13] + w2 * ca[14] + w3 * ca[15]) * ptz
                vv = (w1 * cb[0] + w2 * cb[1] + w3 * cb[2]) * ptz
                upd = (inb & (w1 >= 0.0) & (w2 >= 0.0) & (a0 >= 0.0)
                       & (ptz > bs))
                bs = jnp.where(upd, ptz, bs)
                bu = jnp.where(upd, uu, bu)
                bv = jnp.where(upd, vv, bv)

            obuf[0, :] = bu
            obuf[1, :] = bv
            obuf[2, :] = bs
            pltpu.sync_copy(obuf, out_hbm.at[wid])

    return project


def kernel(vertices, points, faces, uv, uvfaces):
    T = faces.shape[0]
    P = points.shape[0]
    NV = vertices.shape[0]
    NU = uv.shape[0]
    tpad = -(-T // _L) * _L
    pad = tpad - T

    f = faces.astype(jnp.int32)
    uf = uvfaces.astype(jnp.int32)

    def pc(col):
        return jnp.pad(col, (0, pad))

    ci = jnp.concatenate([pc(f[:, 0]), pc(f[:, 1]), pc(f[:, 2]),
                          pc(uf[:, 0]), pc(uf[:, 1]), pc(uf[:, 2])])
    cf = jnp.concatenate([vertices[:, 0], vertices[:, 1], vertices[:, 2],
                          uv[:, 0], uv[:, 1]])
    pts = points.T.reshape(2, P // _L, _L).transpose(1, 0, 2)

    out = _make_project(T, P, NV, NU)(cf, ci, pts)
    return out.transpose(0, 2, 1).reshape(P, 3)
